# Initial kernel scaffold; baseline (speedup 1.0000x reference)
import jax
import jax.numpy as jnp
from jax.experimental import pallas as pl


def kernel(pred, target):
    raise NotImplementedError("write your pallas kernel here")


# speedup vs baseline: 1.0758x; 1.0758x over previous
import functools

import jax
import jax.numpy as jnp
from jax import lax
from jax.experimental import pallas as pl
from jax.experimental.pallas import tpu as pltpu

SMOOTHING = 0.1
ROW_BLOCK = 8


def _sweep_kernel(tgt_ref, x_ref, out_ref):
    i = pl.program_id(0)

    @pl.when(i == 0)
    def _init():
        out_ref[0, 0] = 0.0

    x = x_ref[...]
    rows, cols = x.shape
    t = tgt_ref[pl.ds(i * rows, rows), :]
    col_ids = lax.broadcasted_iota(jnp.int32, (rows, cols), 1)
    hit = col_ids == t
    y = jnp.maximum(x, 0.0) - SMOOTHING * x + jnp.log1p(jnp.exp(-jnp.abs(x)))
    y = y - jnp.where(hit, (1.0 - 2.0 * SMOOTHING) * x, 0.0)
    out_ref[0, 0] += jnp.sum(y)


@functools.partial(jax.jit, static_argnames=("interpret",))
def kernel(pred, target, interpret: bool = False):
    b, c = pred.shape
    tgt = target.astype(jnp.int32).reshape(b, 1)
    grid = (b // ROW_BLOCK,)
    total = pl.pallas_call(
        _sweep_kernel,
        grid=grid,
        in_specs=[
            pl.BlockSpec((b, 1), lambda i: (0, 0)),
            pl.BlockSpec((ROW_BLOCK, c), lambda i: (i, 0)),
        ],
        out_specs=pl.BlockSpec(memory_space=pltpu.SMEM),
        out_shape=jax.ShapeDtypeStruct((1, 1), jnp.float32),
        interpret=interpret,
    )(tgt, pred)
    return (total[0, 0] / (b * c)).astype(pred.dtype)


# speedup vs baseline: 1.3916x; 1.2935x over previous
import functools

import jax
import jax.numpy as jnp
from jax import lax
from jax.experimental import pallas as pl
from jax.experimental.pallas import tpu as pltpu

SMOOTHING = 0.1
ROW_BLOCK = 8
LOG2E = 1.4426950408889634
LN2 = 0.6931471805599453


def _sweep_kernel(tgt_ref, x_ref, out_ref):
    i = pl.program_id(0)
    x = x_ref[...]
    rows, cols = x.shape
    t = tgt_ref[pl.ds(i * rows, rows), :]
    col_ids = lax.broadcasted_iota(jnp.int32, (rows, cols), 1)
    hit = col_ids == t
    u = x * LOG2E
    e = jnp.exp2(jnp.minimum(u, -u))
    lg = jnp.log2(1.0 + e)
    zp = jnp.maximum(x, 0.0)
    xh = jnp.where(hit, x, 0.0)
    s_lg = jnp.sum(lg)
    s_zx = jnp.sum(zp - SMOOTHING * x - (1.0 - 2.0 * SMOOTHING) * xh)
    out_ref[i, 0] = LN2 * s_lg + s_zx


@functools.partial(jax.jit, static_argnames=("interpret",))
def kernel(pred, target, interpret: bool = False):
    b, c = pred.shape
    tgt = target.astype(jnp.int32).reshape(b, 1)
    nb = b // ROW_BLOCK
    partials = pl.pallas_call(
        _sweep_kernel,
        grid=(nb,),
        in_specs=[
            pl.BlockSpec((b, 1), lambda i: (0, 0)),
            pl.BlockSpec((ROW_BLOCK, c), lambda i: (i, 0)),
        ],
        out_specs=pl.BlockSpec(memory_space=pltpu.SMEM),
        out_shape=jax.ShapeDtypeStruct((nb, 1), jnp.float32),
        compiler_params=pltpu.CompilerParams(
            dimension_semantics=("parallel",),
        ),
        interpret=interpret,
    )(tgt, pred)
    return (jnp.sum(partials) / (b * c)).astype(pred.dtype)


# speedup vs baseline: 1.4004x; 1.0064x over previous
import functools

import jax
import jax.numpy as jnp
from jax import lax
from jax.experimental import pallas as pl
from jax.experimental.pallas import tpu as pltpu

SMOOTHING = 0.1
ROW_BLOCK = 32
LOG2E = 1.4426950408889634
LN2 = 0.6931471805599453


def _sweep_kernel(tgt_ref, x_ref, out_ref):
    i = pl.program_id(0)
    x = x_ref[...]
    rows, cols = x.shape
    t = tgt_ref[pl.ds(i * rows, rows), :]
    col_ids = lax.broadcasted_iota(jnp.int32, (rows, cols), 1)
    hit = col_ids == t
    a = jnp.maximum(x, -x)
    e = jnp.exp2(a * (-LOG2E))
    lg = jnp.log(1.0 + e)
    s_l = jnp.sum(lg)
    s_a = jnp.sum(a)
    s_x = jnp.sum(x)
    s_h = jnp.sum(jnp.where(hit, x, 0.0))
    out_ref[i, 0] = (s_l + 0.5 * s_a + (0.5 - SMOOTHING) * s_x
                     - (1.0 - 2.0 * SMOOTHING) * s_h)


@functools.partial(jax.jit, static_argnames=("interpret",))
def kernel(pred, target, interpret: bool = False):
    b, c = pred.shape
    tgt = target.astype(jnp.int32).reshape(b, 1)
    nb = b // ROW_BLOCK
    partials = pl.pallas_call(
        _sweep_kernel,
        grid=(nb,),
        in_specs=[
            pl.BlockSpec((b, 1), lambda i: (0, 0)),
            pl.BlockSpec((ROW_BLOCK, c), lambda i: (i, 0)),
        ],
        out_specs=pl.BlockSpec(memory_space=pltpu.SMEM),
        out_shape=jax.ShapeDtypeStruct((nb, 1), jnp.float32),
        compiler_params=pltpu.CompilerParams(
            dimension_semantics=("parallel",),
        ),
        interpret=interpret,
    )(tgt, pred)
    return (jnp.sum(partials) / (b * c)).astype(pred.dtype)
